# baseline (speedup 1.0000x reference)
import jax
import jax.numpy as jnp
from jax.experimental import pallas as pl


def _kl_kernel(mean_ref, logvar_ref, out_ref):
    m = mean_ref[...]
    lv = logvar_ref[...]
    out_ref[...] = jnp.reshape(-0.5 * jnp.mean(jnp.sum(1.0 + lv - m ** 2 - jnp.exp(lv), axis=-1)), (1, 1))


def kernel(pieces, atom_ids, atom_piece_ids, atom_pos_ids, edge_index, edge_type, node_graph_ids, pred_edge_index, cond, eps, atom_emb, piece_emb, pos_emb, W_mean, b_mean, W_logvar, b_logvar, W_l2h, b_l2h, gru_Wih, gru_Whh, gru_bih, gru_bhh, W_vocab, b_vocab, lin_in_W, lin_in_b, edge_emb, gine_eps, gine_W1, gine_b1, gine_W2, gine_b2, We1, be1, We2, be2, We3, be3, We4, be4):
    N = atom_ids.shape[0]
    T = gine_eps.shape[0]
    mean = cond @ W_mean + b_mean
    log_var = cond @ W_logvar + b_logvar
    z = mean + jnp.exp(0.5 * log_var) * eps
    kl = pl.pallas_call(
        _kl_kernel,
        out_shape=jax.ShapeDtypeStruct((1, 1), jnp.float32),
    )(mean, log_var)[0, 0]
    h0 = z @ W_l2h + b_l2h
    xseq = jnp.transpose(piece_emb[pieces], (1, 0, 2))

    def step(h, x_t):
        gi = x_t @ gru_Wih.T + gru_bih
        gh = h @ gru_Whh.T + gru_bhh
        i_r, i_z, i_n = jnp.split(gi, 3, axis=-1)
        h_r, h_z, h_n = jnp.split(gh, 3, axis=-1)
        r = jax.nn.sigmoid(i_r + h_r)
        u = jax.nn.sigmoid(i_z + h_z)
        n = jnp.tanh(i_n + r * h_n)
        h_new = (1.0 - u) * n + u * h
        return h_new, h_new

    _, hs = jax.lax.scan(step, h0, xseq)
    piece_logits = jnp.transpose(hs, (1, 0, 2)) @ W_vocab + b_vocab
    node_x = jnp.concatenate([atom_emb[atom_ids], piece_emb[atom_piece_ids], pos_emb[atom_pos_ids]], axis=-1)
    h = jax.nn.relu(node_x @ lin_in_W + lin_in_b)
    src = edge_index[0]
    dst = edge_index[1]
    e_feat = edge_emb[edge_type]
    for i in range(T):
        msg = jax.nn.relu(h[src] + e_feat)
        agg = jax.ops.segment_sum(msg, dst, num_segments=N)
        h_in = (1.0 + gine_eps[i]) * h + agg
        h = jax.nn.relu(h_in @ gine_W1[i] + gine_b1[i])
        h = jax.nn.relu(h @ gine_W2[i] + gine_b2[i])
    ps = pred_edge_index[0]
    pd = pred_edge_index[1]
    z_edge = z[node_graph_ids[ps]]
    feat = jnp.concatenate([h[ps], h[pd], z_edge], axis=-1)
    y = jax.nn.relu(feat @ We1 + be1)
    y = jax.nn.relu(y @ We2 + be2)
    y = jax.nn.relu(y @ We3 + be3)
    edge_logits = y @ We4 + be4
    return piece_logits, edge_logits, kl


# speedup vs baseline: 3.6913x; 3.6913x over previous
import functools

import jax
import jax.numpy as jnp
from jax import lax
from jax.experimental import pallas as pl
from jax.experimental.pallas import tpu as pltpu
from jax.experimental.pallas import tpu_sc as plsc

N = 50000
E = 800000
P = 200000
B = 128
L = 64
HR = 128
V = 800
DH = 64
NET = 4
MLP_H = 96

NC, NS = 2, 16
NW = NC * NS

HALF = N // 2
DROW = HALF
SROWS = 26624
E_ROWS = 6272
E2 = E_ROWS * 128
TROWS_E = E_ROWS // NS
EK = 2

N2 = NW * 1664
NTCH = 1664 // 128

P2 = 204800
P_ROWS = P2 // 128
PW = P_ROWS // NW
PK = 5

_sc_mesh = plsc.VectorSubcoreMesh(core_axis_name="c", subcore_axis_name="s")


def _sigmoid(x):
    return 1.0 / (1.0 + jnp.exp(-x))


def _relu(x):
    return jnp.maximum(x, 0.0)


def _dot(a, b):
    return jnp.dot(a, b, preferred_element_type=jnp.float32)


def _dec_body(p_ref, pemb_ref, wih_ref, bih_ref, whh_ref, bhh_ref,
              cond_ref, eps_ref, wm_ref, bm_ref, wlv_ref, blv_ref,
              wl2h_ref, bl2h_ref, wvoc_ref, bvoc_ref,
              logits_ref, z_ref, kl_ref, h_ref):
    i = pl.program_id(0)

    @pl.when(i == 0)
    def _():
        cond = cond_ref[...]
        mean = _dot(cond, wm_ref[...]) + bm_ref[...]
        lv = _dot(cond, wlv_ref[...]) + blv_ref[...]
        z = mean + jnp.exp(0.5 * lv) * eps_ref[...]
        z_ref[...] = z
        kl_ref[0, 0] = -0.5 * jnp.mean(
            jnp.sum(1.0 + lv - mean * mean - jnp.exp(lv), axis=-1))
        h_ref[...] = _dot(z, wl2h_ref[...]) + bl2h_ref[...]

    pc = p_ref[0, 0, :]
    oh = (pc[:, None] == lax.broadcasted_iota(jnp.int32, (B, V), 1)
          ).astype(jnp.float32)
    x = _dot(oh, pemb_ref[...])
    gi = _dot(x, wih_ref[...]) + bih_ref[...]
    h = h_ref[...]
    gh = _dot(h, whh_ref[...]) + bhh_ref[...]
    r = _sigmoid(gi[:, 0:HR] + gh[:, 0:HR])
    u = _sigmoid(gi[:, HR:2 * HR] + gh[:, HR:2 * HR])
    n = jnp.tanh(gi[:, 2 * HR:] + r * gh[:, 2 * HR:])
    hn = (1.0 - u) * n + u * h
    h_ref[...] = hn
    logits_ref[...] = (_dot(hn, wvoc_ref[...]) + bvoc_ref[...])[None]


def _const2(shape):
    return pl.BlockSpec(shape, lambda i: (0, 0))


_decoder = pl.pallas_call(
    _dec_body,
    grid=(L,),
    in_specs=[
        pl.BlockSpec((1, 1, B), lambda i: (i, 0, 0)),
        _const2((V, 32)), _const2((32, 3 * HR)), _const2((1, 3 * HR)),
        _const2((HR, 3 * HR)), _const2((1, 3 * HR)),
        _const2((B, 64)), _const2((B, 64)),
        _const2((64, 64)), _const2((1, 64)), _const2((64, 64)), _const2((1, 64)),
        _const2((64, HR)), _const2((1, HR)), _const2((HR, V)), _const2((1, V)),
    ],
    out_specs=[
        pl.BlockSpec((1, B, V), lambda i: (i, 0, 0)),
        _const2((B, 64)),
        pl.BlockSpec((1, 1), lambda i: (0, 0), memory_space=pltpu.SMEM),
    ],
    out_shape=[
        jax.ShapeDtypeStruct((L, B, V), jnp.float32),
        jax.ShapeDtypeStruct((B, 64), jnp.float32),
        jax.ShapeDtypeStruct((1, 1), jnp.float32),
    ],
    scratch_shapes=[pltpu.VMEM((B, HR), jnp.float32)],
)


def _prep_body(aemb_ref, pemb_ref, posemb_ref, w_ref, z_ref, we1_ref, be1_ref,
               ta_ref, tp_ref, tpos_ref, zc_ref):
    ta_ref[...] = _dot(aemb_ref[...], w_ref[0:32, :])
    tp_ref[...] = _dot(pemb_ref[...], w_ref[32:64, :])
    tpos_ref[...] = _dot(posemb_ref[...], w_ref[64:80, :])
    zc_ref[...] = _dot(z_ref[...], we1_ref[128:192, :]) + be1_ref[...]


_prep = pl.pallas_call(
    _prep_body,
    out_shape=[
        jax.ShapeDtypeStruct((120, DH), jnp.float32),
        jax.ShapeDtypeStruct((V, DH), jnp.float32),
        jax.ShapeDtypeStruct((256, DH), jnp.float32),
        jax.ShapeDtypeStruct((B, MLP_H), jnp.float32),
    ],
)


@functools.partial(
    pl.kernel,
    out_type=jax.ShapeDtypeStruct((N2, DH), jnp.float32),
    mesh=_sc_mesh,
    compiler_params=pltpu.CompilerParams(use_tc_tiling_on_sc=False),
    scratch_types=[
        pltpu.VMEM((128,), jnp.int32),
        pltpu.VMEM((128, DH), jnp.float32),
        pltpu.SemaphoreType.DMA,
    ],
)
def _sc_embed(aid_hbm, pid_hbm, pos_hbm, ta_hbm, tp_hbm, tpos_hbm,
              out_hbm, idx_v, rows_v, sem):
    wid = lax.axis_index("s") * NC + lax.axis_index("c")
    base0 = wid * (N2 // NW)

    @pl.loop(0, NTCH)
    def _chunk(c):
        base = base0 + c * 128
        pltpu.sync_copy(aid_hbm.at[pl.ds(base, 128)], idx_v)
        pltpu.async_copy(ta_hbm.at[idx_v], rows_v, sem).wait()
        pltpu.sync_copy(pid_hbm.at[pl.ds(base, 128)], idx_v)
        pltpu.async_copy(tp_hbm.at[idx_v], rows_v, sem, add=True).wait()
        pltpu.sync_copy(pos_hbm.at[pl.ds(base, 128)], idx_v)
        pltpu.async_copy(tpos_hbm.at[idx_v], rows_v, sem, add=True).wait()
        pltpu.sync_copy(rows_v, out_hbm.at[pl.ds(base, 128)])


def _init_body(pre_ref, b_ref, ee_ref, h_ref, t4_ref):
    h = _relu(pre_ref[...] + b_ref[...])
    h_ref[...] = h
    t4_ref[...] = _relu(h[:, None, :] + ee_ref[...][None, :, :])


_init = pl.pallas_call(
    _init_body,
    grid=(25,),
    in_specs=[
        pl.BlockSpec((2000, DH), lambda i: (i, 0)),
        _const2((1, DH)),
        _const2((NET, DH)),
    ],
    out_specs=[
        pl.BlockSpec((2000, DH), lambda i: (i, 0)),
        pl.BlockSpec((2000, NET, DH), lambda i: (i, 0, 0)),
    ],
    out_shape=[
        jax.ShapeDtypeStruct((N, DH), jnp.float32),
        jax.ShapeDtypeStruct((N, NET, DH), jnp.float32),
    ],
)


@functools.partial(
    pl.kernel,
    out_type=jax.ShapeDtypeStruct((NC, SROWS, DH), jnp.float32),
    mesh=_sc_mesh,
    compiler_params=pltpu.CompilerParams(use_tc_tiling_on_sc=False),
    scratch_types=[
        pltpu.VMEM((EK, 128), jnp.int32),
        pltpu.VMEM((EK, 128), jnp.int32),
        pltpu.VMEM((EK, 128, DH), jnp.float32),
        pltpu.VMEM_SHARED((SROWS, DH), jnp.float32),
        pltpu.SemaphoreType.DMA,
        pltpu.SemaphoreType.DMA,
    ],
)
def _sc_segsum(t4_hbm, src_hbm, dl_hbm, z128_hbm, out_hbm,
               idx_s, idx_d, rows, shared, sg, ss):
    core = lax.axis_index("c")
    s = lax.axis_index("s")
    rbase = s * (SROWS // NS)

    @pl.loop(0, SROWS // NS // 128)
    def _zero(c):
        pltpu.sync_copy(z128_hbm, shared.at[pl.ds(rbase + c * 128, 128)])

    plsc.subcore_barrier()

    ebase = s * TROWS_E

    @pl.loop(0, TROWS_E // EK)
    def _chunk(c):
        crow = ebase + c * EK
        pltpu.sync_copy(src_hbm.at[pl.ds(crow, EK)], idx_s)
        pltpu.sync_copy(dl_hbm.at[core, pl.ds(crow, EK)], idx_d)
        gs = [pltpu.async_copy(t4_hbm.at[idx_s.at[j]], rows.at[j], sg)
              for j in range(EK)]
        for g in gs:
            g.wait()
        scs = [pltpu.async_copy(rows.at[j], shared.at[idx_d.at[j]], ss, add=True)
               for j in range(EK)]
        for d in scs:
            d.wait()

    plsc.subcore_barrier()

    @pl.loop(0, SROWS // NS // 128)
    def _out(c):
        r0 = rbase + c * 128
        pltpu.sync_copy(shared.at[pl.ds(r0, 128)], out_hbm.at[core, pl.ds(r0, 128)])


def _mid_body(h_ref, agg_ref, eps_ref, w1_ref, b1_ref, w2_ref, b2_ref, ee_ref,
              h2_ref, t4_ref):
    e = eps_ref[0, 0]
    hin = (1.0 + e) * h_ref[...] + agg_ref[0]
    y = _relu(_dot(hin, w1_ref[...]) + b1_ref[...])
    h2 = _relu(_dot(y, w2_ref[...]) + b2_ref[...])
    h2_ref[...] = h2
    t4_ref[...] = _relu(h2[:, None, :] + ee_ref[...][None, :, :])


def _last_body(h_ref, agg_ref, eps_ref, w1_ref, b1_ref, w2_ref, b2_ref,
               gid_ref, zc_ref, we1_ref, a_ref, bb_ref):
    e = eps_ref[0, 0]
    hin = (1.0 + e) * h_ref[...] + agg_ref[0]
    y = _relu(_dot(hin, w1_ref[...]) + b1_ref[...])
    h2 = _relu(_dot(y, w2_ref[...]) + b2_ref[...])
    gid = gid_ref[0, 0, :]
    oh = (gid[:, None] == lax.broadcasted_iota(jnp.int32, (1000, B), 1)
          ).astype(jnp.float32)
    a_ref[...] = _dot(h2, we1_ref[0:64, :]) + _dot(oh, zc_ref[...])
    bb_ref[...] = _dot(h2, we1_ref[64:128, :])


_upd_common_in = [
    pl.BlockSpec((1000, DH), lambda i: (i, 0)),
    pl.BlockSpec((1, 1000, DH), lambda i: (i // 25, i % 25, 0)),
    pl.BlockSpec((1, 1), lambda i: (0, 0), memory_space=pltpu.SMEM),
    _const2((DH, DH)), _const2((1, DH)), _const2((DH, DH)), _const2((1, DH)),
]

_mid = pl.pallas_call(
    _mid_body,
    grid=(50,),
    in_specs=_upd_common_in + [_const2((NET, DH))],
    out_specs=[
        pl.BlockSpec((1000, DH), lambda i: (i, 0)),
        pl.BlockSpec((1000, NET, DH), lambda i: (i, 0, 0)),
    ],
    out_shape=[
        jax.ShapeDtypeStruct((N, DH), jnp.float32),
        jax.ShapeDtypeStruct((N, NET, DH), jnp.float32),
    ],
)

_last = pl.pallas_call(
    _last_body,
    grid=(50,),
    in_specs=_upd_common_in + [
        pl.BlockSpec((1, 1, 1000), lambda i: (i, 0, 0)),
        _const2((B, MLP_H)),
        _const2((2 * DH + 64, MLP_H)),
    ],
    out_specs=[
        pl.BlockSpec((1000, MLP_H), lambda i: (i, 0)),
        pl.BlockSpec((1000, MLP_H), lambda i: (i, 0)),
    ],
    out_shape=[
        jax.ShapeDtypeStruct((N, MLP_H), jnp.float32),
        jax.ShapeDtypeStruct((N, MLP_H), jnp.float32),
    ],
)


@functools.partial(
    pl.kernel,
    out_type=jax.ShapeDtypeStruct((P2, MLP_H), jnp.float32),
    mesh=_sc_mesh,
    compiler_params=pltpu.CompilerParams(use_tc_tiling_on_sc=False),
    scratch_types=[
        pltpu.VMEM((PK, 128), jnp.int32),
        pltpu.VMEM((PK, 128), jnp.int32),
        pltpu.VMEM((PK, 128, MLP_H), jnp.float32),
        pltpu.SemaphoreType.DMA,
    ],
)
def _sc_edge(a_hbm, b_hbm, ps_hbm, pd_hbm, out_hbm, idx_s, idx_d, rows, sem):
    wid = lax.axis_index("s") * NC + lax.axis_index("c")
    rbase = wid * PW

    @pl.loop(0, PW // PK)
    def _chunk(c):
        crow = rbase + c * PK
        pltpu.sync_copy(ps_hbm.at[pl.ds(crow, PK)], idx_s)
        pltpu.sync_copy(pd_hbm.at[pl.ds(crow, PK)], idx_d)
        gs = [pltpu.async_copy(b_hbm.at[idx_d.at[j]], rows.at[j], sem)
              for j in range(PK)]
        for g in gs:
            g.wait()
        ga = [pltpu.async_copy(a_hbm.at[idx_s.at[j]], rows.at[j], sem, add=True)
              for j in range(PK)]
        for g in ga:
            g.wait()
        ws = [pltpu.async_copy(rows.at[j],
                               out_hbm.at[pl.ds((crow + j) * 128, 128)], sem)
              for j in range(PK)]
        for w in ws:
            w.wait()


def _mlp_body(y_ref, w2_ref, b2_ref, w3_ref, b3_ref, w4_ref, b4_ref, out_ref):
    y = _relu(y_ref[...])
    y = _relu(_dot(y, w2_ref[...]) + b2_ref[...])
    y = _relu(_dot(y, w3_ref[...]) + b3_ref[...])
    out_ref[...] = _dot(y, w4_ref[...]) + b4_ref[...]


_mlp = pl.pallas_call(
    _mlp_body,
    grid=(100,),
    in_specs=[
        pl.BlockSpec((2048, MLP_H), lambda i: (i, 0)),
        _const2((MLP_H, MLP_H)), _const2((1, MLP_H)),
        _const2((MLP_H, MLP_H)), _const2((1, MLP_H)),
        _const2((MLP_H, NET)), _const2((1, NET)),
    ],
    out_specs=pl.BlockSpec((2048, NET), lambda i: (i, 0)),
    out_shape=jax.ShapeDtypeStruct((P2, NET), jnp.float32),
)


def kernel(pieces, atom_ids, atom_piece_ids, atom_pos_ids, edge_index,
           edge_type, node_graph_ids, pred_edge_index, cond, eps, atom_emb,
           piece_emb, pos_emb, W_mean, b_mean, W_logvar, b_logvar, W_l2h,
           b_l2h, gru_Wih, gru_Whh, gru_bih, gru_bhh, W_vocab, b_vocab,
           lin_in_W, lin_in_b, edge_emb, gine_eps, gine_W1, gine_b1, gine_W2,
           gine_b2, We1, be1, We2, be2, We3, be3, We4, be4):
    i32 = jnp.int32
    r1 = lambda v: v.reshape(1, -1)

    pieces3 = jnp.transpose(pieces.astype(i32), (1, 0)).reshape(L, 1, B)
    logits_lbv, z, kl2 = _decoder(
        pieces3, piece_emb, gru_Wih.T, r1(gru_bih), gru_Whh.T, r1(gru_bhh),
        cond, eps, W_mean, r1(b_mean), W_logvar, r1(b_logvar),
        W_l2h, r1(b_l2h), W_vocab, r1(b_vocab))

    ta, tp, tpos, zc = _prep(atom_emb, piece_emb, pos_emb, lin_in_W, z,
                             We1, r1(be1))

    pad_n = lambda v: jnp.pad(v.astype(i32), (0, N2 - N))
    pre = _sc_embed(pad_n(atom_ids), pad_n(atom_piece_ids),
                    pad_n(atom_pos_ids), ta, tp, tpos)[:N]

    h, t4 = _init(pre, r1(lin_in_b), edge_emb)

    src4 = edge_index[0].astype(i32) * NET + edge_type.astype(i32)
    src4p = jnp.pad(src4, (0, E2 - E)).reshape(E_ROWS, 128)
    dst = edge_index[1].astype(i32)
    dl = jnp.stack([
        jnp.where(dst < HALF, dst, DROW),
        jnp.where(dst >= HALF, dst - HALF, DROW),
    ])
    dlp = jnp.pad(dl, ((0, 0), (0, E2 - E)),
                  constant_values=DROW).reshape(NC, E_ROWS, 128)
    z128 = jnp.zeros((128, DH), jnp.float32)
    gid3 = node_graph_ids.astype(i32).reshape(50, 1, 1000)

    for t in range(4):
        agg2 = _sc_segsum(t4.reshape(N * NET, DH), src4p, dlp, z128)
        epst = jnp.reshape(gine_eps[t], (1, 1))
        if t < 3:
            h, t4 = _mid(h, agg2, epst, gine_W1[t], r1(gine_b1[t]),
                         gine_W2[t], r1(gine_b2[t]), edge_emb)
        else:
            a_tab, b_tab = _last(h, agg2, epst, gine_W1[t], r1(gine_b1[t]),
                                 gine_W2[t], r1(gine_b2[t]), gid3, zc, We1)

    pad_p = lambda v: jnp.pad(v.astype(i32), (0, P2 - P)).reshape(P_ROWS, 128)
    y1pre = _sc_edge(a_tab, b_tab, pad_p(pred_edge_index[0]),
                     pad_p(pred_edge_index[1]))
    edge_logits = _mlp(y1pre, We2, r1(be2), We3, r1(be3), We4, r1(be4))[:P]

    piece_logits = jnp.transpose(logits_lbv, (1, 0, 2))
    return piece_logits, edge_logits, kl2[0, 0]


# speedup vs baseline: 4.0449x; 1.0958x over previous
import functools

import jax
import jax.numpy as jnp
from jax import lax
from jax.experimental import pallas as pl
from jax.experimental.pallas import tpu as pltpu
from jax.experimental.pallas import tpu_sc as plsc

N = 50000
E = 800000
P = 200000
B = 128
L = 64
HR = 128
V = 800
DH = 64
NET = 4
MLP_H = 96

NC, NS = 2, 16
NW = NC * NS

HALF = N // 2
DROW = HALF
SROWS = 26624
E_ROWS = 6272
E2 = E_ROWS * 128
TROWS_E = E_ROWS // NS
EK = 2

N2 = NW * 1664
NTCH = 1664 // 128

P2 = 204800
P_ROWS = P2 // 128
PW = P_ROWS // NW
PK = 5

_sc_mesh = plsc.VectorSubcoreMesh(core_axis_name="c", subcore_axis_name="s")


def _sigmoid(x):
    return 1.0 / (1.0 + jnp.exp(-x))


def _relu(x):
    return jnp.maximum(x, 0.0)


def _dot(a, b):
    return jnp.dot(a, b, preferred_element_type=jnp.float32)


def _dec_body(p_ref, pemb_ref, wih_ref, bih_ref, whh_ref, bhh_ref,
              cond_ref, eps_ref, wm_ref, bm_ref, wlv_ref, blv_ref,
              wl2h_ref, bl2h_ref, wvoc_ref, bvoc_ref,
              logits_ref, z_ref, kl_ref, h_ref):
    i = pl.program_id(0)

    @pl.when(i == 0)
    def _():
        cond = cond_ref[...]
        mean = _dot(cond, wm_ref[...]) + bm_ref[...]
        lv = _dot(cond, wlv_ref[...]) + blv_ref[...]
        z = mean + jnp.exp(0.5 * lv) * eps_ref[...]
        z_ref[...] = z
        kl_ref[0, 0] = -0.5 * jnp.mean(
            jnp.sum(1.0 + lv - mean * mean - jnp.exp(lv), axis=-1))
        h_ref[...] = _dot(z, wl2h_ref[...]) + bl2h_ref[...]

    pc = p_ref[0, 0, :]
    oh = (pc[:, None] == lax.broadcasted_iota(jnp.int32, (B, V), 1)
          ).astype(jnp.float32)
    x = _dot(oh, pemb_ref[...])
    gi = _dot(x, wih_ref[...]) + bih_ref[...]
    h = h_ref[...]
    gh = _dot(h, whh_ref[...]) + bhh_ref[...]
    r = _sigmoid(gi[:, 0:HR] + gh[:, 0:HR])
    u = _sigmoid(gi[:, HR:2 * HR] + gh[:, HR:2 * HR])
    n = jnp.tanh(gi[:, 2 * HR:] + r * gh[:, 2 * HR:])
    hn = (1.0 - u) * n + u * h
    h_ref[...] = hn
    logits_ref[...] = (_dot(hn, wvoc_ref[...]) + bvoc_ref[...])[None]


def _const2(shape):
    return pl.BlockSpec(shape, lambda i: (0, 0))


_decoder = pl.pallas_call(
    _dec_body,
    grid=(L,),
    in_specs=[
        pl.BlockSpec((1, 1, B), lambda i: (i, 0, 0)),
        _const2((V, 32)), _const2((32, 3 * HR)), _const2((1, 3 * HR)),
        _const2((HR, 3 * HR)), _const2((1, 3 * HR)),
        _const2((B, 64)), _const2((B, 64)),
        _const2((64, 64)), _const2((1, 64)), _const2((64, 64)), _const2((1, 64)),
        _const2((64, HR)), _const2((1, HR)), _const2((HR, V)), _const2((1, V)),
    ],
    out_specs=[
        pl.BlockSpec((1, B, V), lambda i: (i, 0, 0)),
        _const2((B, 64)),
        pl.BlockSpec((1, 1), lambda i: (0, 0), memory_space=pltpu.SMEM),
    ],
    out_shape=[
        jax.ShapeDtypeStruct((L, B, V), jnp.float32),
        jax.ShapeDtypeStruct((B, 64), jnp.float32),
        jax.ShapeDtypeStruct((1, 1), jnp.float32),
    ],
    scratch_shapes=[pltpu.VMEM((B, HR), jnp.float32)],
)


def _prep_body(aemb_ref, pemb_ref, posemb_ref, w_ref, z_ref, we1_ref, be1_ref,
               ta_ref, tp_ref, tpos_ref, zc_ref):
    ta_ref[...] = _dot(aemb_ref[...], w_ref[0:32, :])
    tp_ref[...] = _dot(pemb_ref[...], w_ref[32:64, :])
    tpos_ref[...] = _dot(posemb_ref[...], w_ref[64:80, :])
    zc_ref[...] = _dot(z_ref[...], we1_ref[128:192, :]) + be1_ref[...]


_prep = pl.pallas_call(
    _prep_body,
    out_shape=[
        jax.ShapeDtypeStruct((120, DH), jnp.float32),
        jax.ShapeDtypeStruct((V, DH), jnp.float32),
        jax.ShapeDtypeStruct((256, DH), jnp.float32),
        jax.ShapeDtypeStruct((B, MLP_H), jnp.float32),
    ],
)


@functools.partial(
    pl.kernel,
    out_type=jax.ShapeDtypeStruct((N2, DH), jnp.float32),
    mesh=_sc_mesh,
    compiler_params=pltpu.CompilerParams(use_tc_tiling_on_sc=False),
    scratch_types=[
        pltpu.VMEM((128,), jnp.int32),
        pltpu.VMEM((128, DH), jnp.float32),
        pltpu.SemaphoreType.DMA,
    ],
)
def _sc_embed(aid_hbm, pid_hbm, pos_hbm, ta_hbm, tp_hbm, tpos_hbm,
              out_hbm, idx_v, rows_v, sem):
    wid = lax.axis_index("s") * NC + lax.axis_index("c")
    base0 = wid * (N2 // NW)

    @pl.loop(0, NTCH)
    def _chunk(c):
        base = base0 + c * 128
        pltpu.sync_copy(aid_hbm.at[pl.ds(base, 128)], idx_v)
        pltpu.async_copy(ta_hbm.at[idx_v], rows_v, sem).wait()
        pltpu.sync_copy(pid_hbm.at[pl.ds(base, 128)], idx_v)
        pltpu.async_copy(tp_hbm.at[idx_v], rows_v, sem, add=True).wait()
        pltpu.sync_copy(pos_hbm.at[pl.ds(base, 128)], idx_v)
        pltpu.async_copy(tpos_hbm.at[idx_v], rows_v, sem, add=True).wait()
        pltpu.sync_copy(rows_v, out_hbm.at[pl.ds(base, 128)])


def _init_body(pre_ref, b_ref, ee_ref, h_ref, t4_ref):
    h = _relu(pre_ref[...] + b_ref[...])
    h_ref[...] = h
    t4_ref[...] = _relu(h[:, None, :] + ee_ref[...][None, :, :])


_init = pl.pallas_call(
    _init_body,
    grid=(25,),
    in_specs=[
        pl.BlockSpec((2000, DH), lambda i: (i, 0)),
        _const2((1, DH)),
        _const2((NET, DH)),
    ],
    out_specs=[
        pl.BlockSpec((2000, DH), lambda i: (i, 0)),
        pl.BlockSpec((2000, NET, DH), lambda i: (i, 0, 0)),
    ],
    out_shape=[
        jax.ShapeDtypeStruct((N, DH), jnp.float32),
        jax.ShapeDtypeStruct((N, NET, DH), jnp.float32),
    ],
)


@functools.partial(
    pl.kernel,
    out_type=jax.ShapeDtypeStruct((NC, SROWS, DH), jnp.float32),
    mesh=_sc_mesh,
    compiler_params=pltpu.CompilerParams(use_tc_tiling_on_sc=False),
    scratch_types=[
        pltpu.VMEM((2, 128), jnp.int32),
        pltpu.VMEM((2, 128), jnp.int32),
        pltpu.VMEM((128, DH), jnp.float32),
        pltpu.VMEM((128, DH), jnp.float32),
        pltpu.VMEM_SHARED((SROWS, DH), jnp.float32),
        pltpu.SemaphoreType.DMA,
        pltpu.SemaphoreType.DMA,
        pltpu.SemaphoreType.DMA,
        pltpu.SemaphoreType.DMA,
        pltpu.SemaphoreType.DMA,
    ],
)
def _sc_segsum(t4_hbm, cb_hbm, z128_hbm, out_hbm,
               cb0, cb1, rows0, rows1, shared, g0, g1, s0, s1, zd):
    core = lax.axis_index("c")
    s = lax.axis_index("s")
    rbase = s * (SROWS // NS)

    zcs = [pltpu.async_copy(z128_hbm, shared.at[pl.ds(rbase + c * 128, 128)], zd)
           for c in range(SROWS // NS // 128)]
    for d in zcs:
        d.wait()
    plsc.subcore_barrier()

    ebase = s * TROWS_E
    bufs = ((cb0, rows0, g0, s0), (cb1, rows1, g1, s1))
    for b, (cb, rows, g, _) in enumerate(bufs):
        pltpu.sync_copy(cb_hbm.at[core, ebase + b], cb)
        pltpu.async_copy(t4_hbm.at[cb.at[0]], rows, g)

    def _finish(cb, rows, g, ss):
        pltpu.make_async_copy(t4_hbm.at[cb.at[0]], rows, g).wait()
        pltpu.async_copy(rows, shared.at[cb.at[1]], ss, add=True)
        pltpu.make_async_copy(rows, shared.at[cb.at[1]], ss).wait()

    @pl.loop(0, TROWS_E // 2 - 1)
    def _pipe(o):
        for b, (cb, rows, g, ss) in enumerate(bufs):
            _finish(cb, rows, g, ss)
            pltpu.sync_copy(cb_hbm.at[core, ebase + 2 * o + b + 2], cb)
            pltpu.async_copy(t4_hbm.at[cb.at[0]], rows, g)

    for b, (cb, rows, g, ss) in enumerate(bufs):
        _finish(cb, rows, g, ss)

    plsc.subcore_barrier()
    dcs = []
    for c in range(SROWS // NS // 128):
        r0 = rbase + c * 128
        dcs.append(pltpu.async_copy(shared.at[pl.ds(r0, 128)],
                                    out_hbm.at[core, pl.ds(r0, 128)], zd))
    for d in dcs:
        d.wait()


def _mid_body(h_ref, agg_ref, eps_ref, w1_ref, b1_ref, w2_ref, b2_ref, ee_ref,
              h2_ref, t4_ref):
    e = eps_ref[0, 0]
    hin = (1.0 + e) * h_ref[...] + agg_ref[0]
    y = _relu(_dot(hin, w1_ref[...]) + b1_ref[...])
    h2 = _relu(_dot(y, w2_ref[...]) + b2_ref[...])
    h2_ref[...] = h2
    t4_ref[...] = _relu(h2[:, None, :] + ee_ref[...][None, :, :])


def _last_body(h_ref, agg_ref, eps_ref, w1_ref, b1_ref, w2_ref, b2_ref,
               gid_ref, zc_ref, we1_ref, a_ref, bb_ref):
    e = eps_ref[0, 0]
    hin = (1.0 + e) * h_ref[...] + agg_ref[0]
    y = _relu(_dot(hin, w1_ref[...]) + b1_ref[...])
    h2 = _relu(_dot(y, w2_ref[...]) + b2_ref[...])
    gid = gid_ref[0, 0, :]
    oh = (gid[:, None] == lax.broadcasted_iota(jnp.int32, (1000, B), 1)
          ).astype(jnp.float32)
    a_ref[...] = _dot(h2, we1_ref[0:64, :]) + _dot(oh, zc_ref[...])
    bb_ref[...] = _dot(h2, we1_ref[64:128, :])


_upd_common_in = [
    pl.BlockSpec((1000, DH), lambda i: (i, 0)),
    pl.BlockSpec((1, 1000, DH), lambda i: (i // 25, i % 25, 0)),
    pl.BlockSpec((1, 1), lambda i: (0, 0), memory_space=pltpu.SMEM),
    _const2((DH, DH)), _const2((1, DH)), _const2((DH, DH)), _const2((1, DH)),
]

_mid = pl.pallas_call(
    _mid_body,
    grid=(50,),
    in_specs=_upd_common_in + [_const2((NET, DH))],
    out_specs=[
        pl.BlockSpec((1000, DH), lambda i: (i, 0)),
        pl.BlockSpec((1000, NET, DH), lambda i: (i, 0, 0)),
    ],
    out_shape=[
        jax.ShapeDtypeStruct((N, DH), jnp.float32),
        jax.ShapeDtypeStruct((N, NET, DH), jnp.float32),
    ],
)

_last = pl.pallas_call(
    _last_body,
    grid=(50,),
    in_specs=_upd_common_in + [
        pl.BlockSpec((1, 1, 1000), lambda i: (i, 0, 0)),
        _const2((B, MLP_H)),
        _const2((2 * DH + 64, MLP_H)),
    ],
    out_specs=[
        pl.BlockSpec((1000, MLP_H), lambda i: (i, 0)),
        pl.BlockSpec((1000, MLP_H), lambda i: (i, 0)),
    ],
    out_shape=[
        jax.ShapeDtypeStruct((N, MLP_H), jnp.float32),
        jax.ShapeDtypeStruct((N, MLP_H), jnp.float32),
    ],
)


@functools.partial(
    pl.kernel,
    out_type=jax.ShapeDtypeStruct((P2, MLP_H), jnp.float32),
    mesh=_sc_mesh,
    compiler_params=pltpu.CompilerParams(use_tc_tiling_on_sc=False),
    scratch_types=[
        pltpu.VMEM((PK, 2, 128), jnp.int32),
        pltpu.VMEM((PK, 2, 128), jnp.int32),
        pltpu.VMEM((PK, 128, MLP_H), jnp.float32),
        pltpu.VMEM((PK, 128, MLP_H), jnp.float32),
        pltpu.SemaphoreType.DMA,
        pltpu.SemaphoreType.DMA,
        pltpu.SemaphoreType.DMA,
        pltpu.SemaphoreType.DMA,
    ],
)
def _sc_edge(a_hbm, b_hbm, cb_hbm, out_hbm, cb0, cb1, rows0, rows1,
             g0, g1, w0, w1):
    wid = lax.axis_index("s") * NC + lax.axis_index("c")
    rbase = wid * PW
    bufs = ((cb0, rows0, g0, w0), (cb1, rows1, g1, w1))

    def _start(cb, rows, g, crow):
        pltpu.sync_copy(cb_hbm.at[pl.ds(crow, PK)], cb)
        for j in range(PK):
            pltpu.async_copy(b_hbm.at[cb.at[j, 1]], rows.at[j], g)

    def _finish(cb, rows, g, w, crow):
        for j in range(PK):
            pltpu.make_async_copy(b_hbm.at[cb.at[j, 1]], rows.at[j], g).wait()
        ga = [pltpu.async_copy(a_hbm.at[cb.at[j, 0]], rows.at[j], g, add=True)
              for j in range(PK)]
        for d in ga:
            d.wait()
        ws = [pltpu.async_copy(rows.at[j],
                               out_hbm.at[pl.ds((crow + j) * 128, 128)], w)
              for j in range(PK)]
        for d in ws:
            d.wait()

    for b, (cb, rows, g, _) in enumerate(bufs):
        _start(cb, rows, g, rbase + b * PK)

    @pl.loop(0, PW // PK // 2 - 1)
    def _pipe(o):
        for b, (cb, rows, g, w) in enumerate(bufs):
            crow = rbase + (2 * o + b) * PK
            _finish(cb, rows, g, w, crow)
            _start(cb, rows, g, crow + 2 * PK)

    for b, (cb, rows, g, w) in enumerate(bufs):
        _finish(cb, rows, g, w, rbase + (PW - 2 * PK) + b * PK)


def _mlp_body(y_ref, w2_ref, b2_ref, w3_ref, b3_ref, w4_ref, b4_ref, out_ref):
    y = _relu(y_ref[...])
    y = _relu(_dot(y, w2_ref[...]) + b2_ref[...])
    y = _relu(_dot(y, w3_ref[...]) + b3_ref[...])
    out_ref[...] = _dot(y, w4_ref[...]) + b4_ref[...]


_mlp = pl.pallas_call(
    _mlp_body,
    grid=(100,),
    in_specs=[
        pl.BlockSpec((2048, MLP_H), lambda i: (i, 0)),
        _const2((MLP_H, MLP_H)), _const2((1, MLP_H)),
        _const2((MLP_H, MLP_H)), _const2((1, MLP_H)),
        _const2((MLP_H, NET)), _const2((1, NET)),
    ],
    out_specs=pl.BlockSpec((2048, NET), lambda i: (i, 0)),
    out_shape=jax.ShapeDtypeStruct((P2, NET), jnp.float32),
)


def kernel(pieces, atom_ids, atom_piece_ids, atom_pos_ids, edge_index,
           edge_type, node_graph_ids, pred_edge_index, cond, eps, atom_emb,
           piece_emb, pos_emb, W_mean, b_mean, W_logvar, b_logvar, W_l2h,
           b_l2h, gru_Wih, gru_Whh, gru_bih, gru_bhh, W_vocab, b_vocab,
           lin_in_W, lin_in_b, edge_emb, gine_eps, gine_W1, gine_b1, gine_W2,
           gine_b2, We1, be1, We2, be2, We3, be3, We4, be4):
    i32 = jnp.int32
    r1 = lambda v: v.reshape(1, -1)

    pieces3 = jnp.transpose(pieces.astype(i32), (1, 0)).reshape(L, 1, B)
    logits_lbv, z, kl2 = _decoder(
        pieces3, piece_emb, gru_Wih.T, r1(gru_bih), gru_Whh.T, r1(gru_bhh),
        cond, eps, W_mean, r1(b_mean), W_logvar, r1(b_logvar),
        W_l2h, r1(b_l2h), W_vocab, r1(b_vocab))

    ta, tp, tpos, zc = _prep(atom_emb, piece_emb, pos_emb, lin_in_W, z,
                             We1, r1(be1))

    pad_n = lambda v: jnp.pad(v.astype(i32), (0, N2 - N))
    pre = _sc_embed(pad_n(atom_ids), pad_n(atom_piece_ids),
                    pad_n(atom_pos_ids), ta, tp, tpos)[:N]

    h, t4 = _init(pre, r1(lin_in_b), edge_emb)

    src4 = edge_index[0].astype(i32) * NET + edge_type.astype(i32)
    src4p = jnp.pad(src4, (0, E2 - E)).reshape(E_ROWS, 128)
    dst = edge_index[1].astype(i32)
    dl = jnp.stack([
        jnp.where(dst < HALF, dst, DROW),
        jnp.where(dst >= HALF, dst - HALF, DROW),
    ])
    dlp = jnp.pad(dl, ((0, 0), (0, E2 - E)),
                  constant_values=DROW).reshape(NC, E_ROWS, 128)
    cb = jnp.stack([jnp.broadcast_to(src4p, (NC, E_ROWS, 128)), dlp], axis=2)
    z128 = jnp.zeros((128, DH), jnp.float32)
    gid3 = node_graph_ids.astype(i32).reshape(50, 1, 1000)

    for t in range(4):
        agg2 = _sc_segsum(t4.reshape(N * NET, DH), cb, z128)
        epst = jnp.reshape(gine_eps[t], (1, 1))
        if t < 3:
            h, t4 = _mid(h, agg2, epst, gine_W1[t], r1(gine_b1[t]),
                         gine_W2[t], r1(gine_b2[t]), edge_emb)
        else:
            a_tab, b_tab = _last(h, agg2, epst, gine_W1[t], r1(gine_b1[t]),
                                 gine_W2[t], r1(gine_b2[t]), gid3, zc, We1)

    pad_p = lambda v: jnp.pad(v.astype(i32), (0, P2 - P)).reshape(P_ROWS, 128)
    pcb = jnp.stack([pad_p(pred_edge_index[0]), pad_p(pred_edge_index[1])],
                    axis=1)
    y1pre = _sc_edge(a_tab, b_tab, pcb)
    edge_logits = _mlp(y1pre, We2, r1(be2), We3, r1(be3), We4, r1(be4))[:P]

    piece_logits = jnp.transpose(logits_lbv, (1, 0, 2))
    return piece_logits, edge_logits, kl2[0, 0]
